# baseline (speedup 1.0000x reference)
import jax
import jax.numpy as jnp
from jax import lax
from jax.experimental import pallas as pl
from jax.experimental.pallas import tpu as pltpu
from jax.experimental.pallas import tpu_sc as plsc

EPS = 1e-09
E = 320000
D = 128
N = 10000
NC = 2
NS = 16
NW = NC * NS
NPAD = 10240
SLAB = NPAD // NS
CW = 16
DW = D + CW
EPT = E // NW
R = 128
NFULL = EPT // R
TR = EPT - NFULL * R
NPAIR = NFULL // 2
NSLAB = SLAB // R


def _body_a(x_hbm, idx_hbm, zfull_hbm, mixed_hbm,
            s0_hbm, s1_hbm, c0_hbm, c1_hbm,
            acc_sh,
            idx0_v, idx1_v, idxt_v, rows0_v, rows1_v,
            ld0_s, ld1_s, sc0_s, sc1_s):
    c = lax.axis_index("c")
    s = lax.axis_index("s")
    idxs = (idx0_v, idx1_v)
    rows = (rows0_v, rows1_v)
    lds = (ld0_s, ld1_s)
    scs = (sc0_s, sc1_s)
    row0 = s * SLAB
    pltpu.sync_copy(zfull_hbm, rows0_v)
    zds = []
    for j in range(NSLAB):
        zds.append(pltpu.async_copy(
            rows0_v, acc_sh.at[pl.ds(row0 + j * R, R), :], sc0_s))
    for d in zds:
        d.wait()
    pltpu.sync_copy(mixed_hbm, rows0_v)
    pltpu.sync_copy(mixed_hbm, rows1_v)
    plsc.subcore_barrier()

    base = (c * NS + s) * EPT

    def start_load(off, b):
        pltpu.async_copy(idx_hbm.at[pl.ds(off, R)], idxs[b], lds[b])
        pltpu.async_copy(x_hbm.at[pl.ds(off, R), :],
                         rows[b].at[:, pl.ds(0, D)], lds[b])

    def wait_load(b):
        pltpu.make_async_copy(idx_hbm.at[pl.ds(0, R)], idxs[b], lds[b]).wait()
        pltpu.make_async_copy(x_hbm.at[pl.ds(0, R), :],
                              rows[b].at[:, pl.ds(0, D)], lds[b]).wait()

    start_load(base, 0)
    start_load(base + R, 1)

    def pair(i, carry):
        for b in range(2):
            ch = 2 * i + b
            wait_load(b)
            d1 = pltpu.async_copy(rows[b], acc_sh.at[idxs[b]], scs[b], add=True)
            d1.wait()

            @pl.when(ch + 2 < NFULL)
            def _():
                start_load(base + (ch + 2) * R, b)

        return carry

    lax.fori_loop(0, NPAIR, pair, 0)
    pltpu.sync_copy(idx_hbm.at[pl.ds(base + NFULL * R, TR)], idxt_v)
    pltpu.sync_copy(x_hbm.at[pl.ds(base + NFULL * R, TR), :],
                    rows0_v.at[pl.ds(0, TR), pl.ds(0, D)])
    pltpu.sync_copy(rows0_v.at[pl.ds(0, TR), :], acc_sh.at[idxt_v], add=True)
    plsc.subcore_barrier()

    def dump_read(j, q):
        r0 = row0 + j * R
        pltpu.async_copy(acc_sh.at[pl.ds(r0, R), :], rows[q], lds[q])

    def dump_read_wait(q):
        pltpu.make_async_copy(acc_sh.at[pl.ds(0, R), :], rows[q], lds[q]).wait()

    def dump_write_wait(q):
        pltpu.make_async_copy(rows[q].at[:, pl.ds(0, D)],
                              s0_hbm.at[pl.ds(0, R), :], scs[q]).wait()
        pltpu.make_async_copy(rows[q].at[:, pl.ds(D, CW)],
                              c0_hbm.at[pl.ds(0, R), :], scs[q]).wait()

    dump_read(0, 0)
    for j in range(NSLAB):
        q = j % 2
        r0 = row0 + j * R
        dump_read_wait(q)
        if j + 1 < NSLAB:
            if j >= 1:
                dump_write_wait(q ^ 1)
            dump_read(j + 1, q ^ 1)

        @pl.when(c == 0)
        def _():
            pltpu.async_copy(rows[q].at[:, pl.ds(0, D)],
                             s0_hbm.at[pl.ds(r0, R), :], scs[q])
            pltpu.async_copy(rows[q].at[:, pl.ds(D, CW)],
                             c0_hbm.at[pl.ds(r0, R), :], scs[q])

        @pl.when(c == 1)
        def _():
            pltpu.async_copy(rows[q].at[:, pl.ds(0, D)],
                             s1_hbm.at[pl.ds(r0, R), :], scs[q])
            pltpu.async_copy(rows[q].at[:, pl.ds(D, CW)],
                             c1_hbm.at[pl.ds(r0, R), :], scs[q])

    dump_write_wait(0)
    dump_write_wait(1)


def _body_b(idx_hbm, s0_hbm, s1_hbm, c0_hbm, c1_hbm, out_hbm,
            pooled_sh, ca_v, cb_v,
            idx0_v, idx1_v, idxt_v, rows0_v, rows1_v, rowst_v,
            ld0_s, ld1_s, g_s, st0_s, st1_s):
    c = lax.axis_index("c")
    s = lax.axis_index("s")
    idxs = (idx0_v, idx1_v)
    rows = (rows0_v, rows1_v)
    lds = (ld0_s, ld1_s)
    sts = (st0_s, st1_s)
    row0 = s * SLAB

    HC = R // 2
    NH = SLAB // HC

    def comb_load(t, q):
        r0 = row0 + t * HC
        h = pl.ds(q * HC, HC)
        pltpu.async_copy(s0_hbm.at[pl.ds(r0, HC), :], rows0_v.at[h, :], lds[q])
        pltpu.async_copy(s1_hbm.at[pl.ds(r0, HC), :], rows1_v.at[h, :], lds[q])
        pltpu.async_copy(c0_hbm.at[pl.ds(r0, HC), :], ca_v.at[h, :], lds[q])
        pltpu.async_copy(c1_hbm.at[pl.ds(r0, HC), :], cb_v.at[h, :], lds[q])

    def comb_wait(q):
        h = pl.ds(q * HC, HC)
        pltpu.make_async_copy(s0_hbm.at[pl.ds(0, HC), :], rows0_v.at[h, :],
                              lds[q]).wait()
        pltpu.make_async_copy(s1_hbm.at[pl.ds(0, HC), :], rows1_v.at[h, :],
                              lds[q]).wait()
        pltpu.make_async_copy(c0_hbm.at[pl.ds(0, HC), :], ca_v.at[h, :],
                              lds[q]).wait()
        pltpu.make_async_copy(c1_hbm.at[pl.ds(0, HC), :], cb_v.at[h, :],
                              lds[q]).wait()

    comb_load(0, 0)

    def comb_pair(i, carry):
        for q in range(2):
            t = 2 * i + q
            comb_wait(q)

            @pl.when(t + 1 < NH)
            def _():
                comb_load(t + 1, q ^ 1)

            def nrow(r, cc):
                rr = q * HC + r
                sv = (ca_v[rr, pl.ds(0, 16)] + cb_v[rr, pl.ds(0, 16)]
                      + jnp.float32(EPS))
                scale = jnp.float32(1.0) / sv
                for k in range(8):
                    sl = pl.ds(k * 16, 16)
                    rows0_v[rr, sl] = (rows0_v[rr, sl] + rows1_v[rr, sl]) * scale
                return cc

            lax.fori_loop(0, HC, nrow, 0)
            pltpu.sync_copy(rows0_v.at[pl.ds(q * HC, HC), :],
                            pooled_sh.at[pl.ds(row0 + t * HC, HC), :])
        return carry

    lax.fori_loop(0, NH // 2, comb_pair, 0)
    plsc.subcore_barrier()

    base = (c * NS + s) * EPT

    def wait_idx(b):
        pltpu.make_async_copy(idx_hbm.at[pl.ds(0, R)], idxs[b], lds[b]).wait()

    def wait_store(b):
        pltpu.make_async_copy(rows[b], out_hbm.at[pl.ds(0, R), :], sts[b]).wait()

    pltpu.async_copy(idx_hbm.at[pl.ds(base, R)], idx0_v, ld0_s)
    pltpu.async_copy(idx_hbm.at[pl.ds(base + R, R)], idx1_v, ld1_s)

    def gpair(i, carry):
        for b in range(2):
            ch = 2 * i + b
            wait_idx(b)

            @pl.when(ch >= 2)
            def _():
                wait_store(b)

            g = pltpu.async_copy(pooled_sh.at[idxs[b]], rows[b], g_s)
            g.wait()
            pltpu.async_copy(rows[b], out_hbm.at[pl.ds(base + ch * R, R), :],
                             sts[b])

            @pl.when(ch + 2 < NFULL)
            def _():
                pltpu.async_copy(idx_hbm.at[pl.ds(base + (ch + 2) * R, R)],
                                 idxs[b], lds[b])

        return carry

    lax.fori_loop(0, NPAIR, gpair, 0)
    pltpu.sync_copy(idx_hbm.at[pl.ds(base + NFULL * R, TR)], idxt_v)
    pltpu.sync_copy(pooled_sh.at[idxt_v], rowst_v)
    pltpu.sync_copy(rowst_v, out_hbm.at[pl.ds(base + NFULL * R, TR), :])
    wait_store(0)
    wait_store(1)


def kernel(input, index):
    mesh = plsc.VectorSubcoreMesh(core_axis_name="c", subcore_axis_name="s",
                                  num_cores=NC, num_subcores=NS)
    f32 = jnp.float32
    zfull = jnp.zeros((R, DW), f32)
    mixed = jnp.concatenate(
        [jnp.zeros((R, D), f32), jnp.ones((R, CW), f32)], axis=1)

    cparams = pltpu.CompilerParams(use_tc_tiling_on_sc=False)
    ka = pl.kernel(
        _body_a,
        compiler_params=cparams,
        out_type=[jax.ShapeDtypeStruct((NPAD, D), f32),
                  jax.ShapeDtypeStruct((NPAD, D), f32),
                  jax.ShapeDtypeStruct((NPAD, CW), f32),
                  jax.ShapeDtypeStruct((NPAD, CW), f32)],
        mesh=mesh,
        scratch_types=[
            pltpu.VMEM_SHARED((NPAD, DW), f32),
            pltpu.VMEM((R,), jnp.int32),
            pltpu.VMEM((R,), jnp.int32),
            pltpu.VMEM((TR,), jnp.int32),
            pltpu.VMEM((R, DW), f32),
            pltpu.VMEM((R, DW), f32),
            pltpu.SemaphoreType.DMA,
            pltpu.SemaphoreType.DMA,
            pltpu.SemaphoreType.DMA,
            pltpu.SemaphoreType.DMA,
        ],
    )
    s0, s1, c0, c1 = ka(input, index, zfull, mixed)

    kb = pl.kernel(
        _body_b,
        compiler_params=cparams,
        out_type=jax.ShapeDtypeStruct((E, D), f32),
        mesh=mesh,
        scratch_types=[
            pltpu.VMEM_SHARED((NPAD, D), f32),
            pltpu.VMEM((R, CW), f32),
            pltpu.VMEM((R, CW), f32),
            pltpu.VMEM((R,), jnp.int32),
            pltpu.VMEM((R,), jnp.int32),
            pltpu.VMEM((TR,), jnp.int32),
            pltpu.VMEM((R, D), f32),
            pltpu.VMEM((R, D), f32),
            pltpu.VMEM((TR, D), f32),
            pltpu.SemaphoreType.DMA,
            pltpu.SemaphoreType.DMA,
            pltpu.SemaphoreType.DMA,
            pltpu.SemaphoreType.DMA,
            pltpu.SemaphoreType.DMA,
        ],
    )
    return kb(index, s0, s1, c0, c1)


# speedup vs baseline: 1.0620x; 1.0620x over previous
import jax
import jax.numpy as jnp
from jax import lax
from jax.experimental import pallas as pl
from jax.experimental.pallas import tpu as pltpu
from jax.experimental.pallas import tpu_sc as plsc

EPS = 1e-09
E = 320000
D = 128
N = 10000
NC = 2
NS = 16
NW = NC * NS
NPAD = 10240
SLAB = NPAD // NS
CW = 16
R = 128
NFULL = 78
EPT = NFULL * R
NPAIR = NFULL // 2
NSLAB = SLAB // R
IDXR = E // R
EXBASE = NW * EPT
NEX = (E - EXBASE) // R
EXW = NW - NEX
IB = 16


def _body_a(x_hbm, idx2_hbm, zrow_hbm, zcnt_hbm, one_hbm,
            s0_hbm, s1_hbm, c0_hbm, c1_hbm,
            acc_sh, cnt_sh, zc_v, ones_v,
            idxv, rows0_v, rows1_v,
            ld0_s, ld1_s, sc0_s, sc1_s):
    c = lax.axis_index("c")
    s = lax.axis_index("s")
    rows = (rows0_v, rows1_v)
    lds = (ld0_s, ld1_s)
    scs = (sc0_s, sc1_s)
    row0 = s * SLAB
    w = c * NS + s
    base = w * EPT
    pltpu.sync_copy(zrow_hbm, rows0_v)
    pltpu.sync_copy(zcnt_hbm, zc_v)
    pltpu.sync_copy(one_hbm, ones_v)
    zds = []
    for j in range(NSLAB):
        zds.append(pltpu.async_copy(
            rows0_v, acc_sh.at[pl.ds(row0 + j * R, R), :], sc0_s))
        zds.append(pltpu.async_copy(
            zc_v, cnt_sh.at[pl.ds(row0 + j * R, R), :], sc1_s))
    for d in zds:
        d.wait()
    plsc.subcore_barrier()

    def start_load(ch, b):
        pltpu.async_copy(x_hbm.at[pl.ds(base + ch * R, R), :], rows[b], lds[b])

    def wait_load(b):
        pltpu.make_async_copy(x_hbm.at[pl.ds(0, R), :], rows[b], lds[b]).wait()

    start_load(0, 0)
    start_load(1, 1)

    def pair(i, carry):
        for b in range(2):
            ch = 2 * i + b
            if b == 0:
                @pl.when(lax.rem(ch, 16) == 0)
                def _():
                    pltpu.sync_copy(idx2_hbm.at[pl.ds(w * NFULL + ch, IB), :],
                                    idxv)
            wait_load(b)
            ix = idxv.at[lax.rem(ch, 16)]
            d1 = pltpu.async_copy(rows[b], acc_sh.at[ix], scs[b], add=True)
            d2 = pltpu.async_copy(ones_v, cnt_sh.at[ix], scs[b], add=True)
            d1.wait()
            d2.wait()

            @pl.when(ch + 2 < NFULL)
            def _():
                start_load(ch + 2, b)

        return carry

    lax.fori_loop(0, NPAIR, pair, 0)

    pltpu.sync_copy(idx2_hbm.at[pl.ds(NW * NFULL, NEX), :],
                    idxv.at[pl.ds(0, NEX), :])

    @pl.when(w >= EXW)
    def _():
        off = EXBASE + (w - EXW) * R
        pltpu.sync_copy(x_hbm.at[pl.ds(off, R), :], rows0_v)
        ix = idxv.at[w - EXW]
        pltpu.sync_copy(rows0_v, acc_sh.at[ix], add=True)
        pltpu.sync_copy(ones_v, cnt_sh.at[ix], add=True)

    plsc.subcore_barrier()

    cbufs = (zc_v, ones_v)

    def dump_read(j, q):
        r0 = row0 + j * R
        pltpu.async_copy(acc_sh.at[pl.ds(r0, R), :], rows[q], lds[q])
        pltpu.async_copy(cnt_sh.at[pl.ds(r0, R), :], cbufs[q], lds[q])

    def dump_read_wait(q):
        pltpu.make_async_copy(acc_sh.at[pl.ds(0, R), :], rows[q], lds[q]).wait()
        pltpu.make_async_copy(cnt_sh.at[pl.ds(0, R), :], cbufs[q], lds[q]).wait()

    def dump_write_wait(q):
        pltpu.make_async_copy(rows[q], s0_hbm.at[pl.ds(0, R), :], scs[q]).wait()
        pltpu.make_async_copy(cbufs[q], c0_hbm.at[pl.ds(0, R), :], scs[q]).wait()

    dump_read(0, 0)
    for j in range(NSLAB):
        q = j % 2
        r0 = row0 + j * R
        dump_read_wait(q)
        if j + 1 < NSLAB:
            if j >= 1:
                dump_write_wait(q ^ 1)
            dump_read(j + 1, q ^ 1)

        @pl.when(c == 0)
        def _():
            pltpu.async_copy(rows[q], s0_hbm.at[pl.ds(r0, R), :], scs[q])
            pltpu.async_copy(cbufs[q], c0_hbm.at[pl.ds(r0, R), :], scs[q])

        @pl.when(c == 1)
        def _():
            pltpu.async_copy(rows[q], s1_hbm.at[pl.ds(r0, R), :], scs[q])
            pltpu.async_copy(cbufs[q], c1_hbm.at[pl.ds(r0, R), :], scs[q])

    dump_write_wait(0)
    dump_write_wait(1)


def _body_b(idx2_hbm, s0_hbm, s1_hbm, c0_hbm, c1_hbm, out_hbm,
            pooled_sh, ca_v, cb_v,
            idxv, rows0_v, rows1_v,
            ld0_s, ld1_s, g_s, st0_s, st1_s):
    c = lax.axis_index("c")
    s = lax.axis_index("s")
    rows = (rows0_v, rows1_v)
    lds = (ld0_s, ld1_s)
    sts = (st0_s, st1_s)
    row0 = s * SLAB
    w = c * NS + s
    base = w * EPT
    pltpu.async_copy(idx2_hbm.at[pl.ds(w * NFULL, NFULL), :],
                     idxv.at[pl.ds(0, NFULL), :], g_s)
    pltpu.async_copy(idx2_hbm.at[pl.ds(NW * NFULL, NEX), :],
                     idxv.at[pl.ds(NFULL, NEX), :], g_s)

    HC = R // 2
    NH = SLAB // HC

    def comb_load(t, q):
        r0 = row0 + t * HC
        h = pl.ds(q * HC, HC)
        pltpu.async_copy(s0_hbm.at[pl.ds(r0, HC), :], rows0_v.at[h, :], lds[q])
        pltpu.async_copy(s1_hbm.at[pl.ds(r0, HC), :], rows1_v.at[h, :], lds[q])
        pltpu.async_copy(c0_hbm.at[pl.ds(r0, HC), :], ca_v.at[h, :], lds[q])
        pltpu.async_copy(c1_hbm.at[pl.ds(r0, HC), :], cb_v.at[h, :], lds[q])

    def comb_wait(q):
        h = pl.ds(q * HC, HC)
        pltpu.make_async_copy(s0_hbm.at[pl.ds(0, HC), :], rows0_v.at[h, :],
                              lds[q]).wait()
        pltpu.make_async_copy(s1_hbm.at[pl.ds(0, HC), :], rows1_v.at[h, :],
                              lds[q]).wait()
        pltpu.make_async_copy(c0_hbm.at[pl.ds(0, HC), :], ca_v.at[h, :],
                              lds[q]).wait()
        pltpu.make_async_copy(c1_hbm.at[pl.ds(0, HC), :], cb_v.at[h, :],
                              lds[q]).wait()

    comb_load(0, 0)

    def comb_pair(i, carry):
        for q in range(2):
            t = 2 * i + q
            comb_wait(q)

            @pl.when(t + 1 < NH)
            def _():
                comb_load(t + 1, q ^ 1)

            def nrow(r, cc):
                rr = q * HC + r
                sv = (ca_v[rr, pl.ds(0, 16)] + cb_v[rr, pl.ds(0, 16)]
                      + jnp.float32(EPS))
                scale = jnp.float32(1.0) / sv
                for k in range(8):
                    sl = pl.ds(k * 16, 16)
                    rows0_v[rr, sl] = (rows0_v[rr, sl] + rows1_v[rr, sl]) * scale
                return cc

            lax.fori_loop(0, HC, nrow, 0)
            pltpu.sync_copy(rows0_v.at[pl.ds(q * HC, HC), :],
                            pooled_sh.at[pl.ds(row0 + t * HC, HC), :])
        return carry

    lax.fori_loop(0, NH // 2, comb_pair, 0)
    pltpu.make_async_copy(idx2_hbm.at[pl.ds(0, NFULL), :],
                          idxv.at[pl.ds(0, NFULL), :], g_s).wait()
    pltpu.make_async_copy(idx2_hbm.at[pl.ds(0, NEX), :],
                          idxv.at[pl.ds(NFULL, NEX), :], g_s).wait()
    plsc.subcore_barrier()

    def wait_store(b):
        pltpu.make_async_copy(rows[b], out_hbm.at[pl.ds(0, R), :], sts[b]).wait()

    def gpair(i, carry):
        for b in range(2):
            ch = 2 * i + b

            @pl.when(ch >= 2)
            def _():
                wait_store(b)

            g = pltpu.async_copy(pooled_sh.at[idxv.at[ch]], rows[b], g_s)
            g.wait()
            pltpu.async_copy(rows[b], out_hbm.at[pl.ds(base + ch * R, R), :],
                             sts[b])

        return carry

    lax.fori_loop(0, NPAIR, gpair, 0)
    wait_store(0)
    wait_store(1)

    @pl.when(w >= EXW)
    def _():
        off = EXBASE + (w - EXW) * R
        pltpu.sync_copy(pooled_sh.at[idxv.at[NFULL + (w - EXW)]], rows0_v)
        pltpu.sync_copy(rows0_v, out_hbm.at[pl.ds(off, R), :])


def kernel(input, index):
    mesh = plsc.VectorSubcoreMesh(core_axis_name="c", subcore_axis_name="s",
                                  num_cores=NC, num_subcores=NS)
    f32 = jnp.float32
    zrow = jnp.zeros((R, D), f32)
    zcnt = jnp.zeros((R, CW), f32)
    ones = jnp.ones((R, CW), f32)
    idx2 = index.reshape(IDXR, R)

    cparams = pltpu.CompilerParams(use_tc_tiling_on_sc=False)
    ka = pl.kernel(
        _body_a,
        compiler_params=cparams,
        out_type=[jax.ShapeDtypeStruct((NPAD, D), f32),
                  jax.ShapeDtypeStruct((NPAD, D), f32),
                  jax.ShapeDtypeStruct((NPAD, CW), f32),
                  jax.ShapeDtypeStruct((NPAD, CW), f32)],
        mesh=mesh,
        scratch_types=[
            pltpu.VMEM_SHARED((NPAD, D), f32),
            pltpu.VMEM_SHARED((NPAD, CW), f32),
            pltpu.VMEM((R, CW), f32),
            pltpu.VMEM((R, CW), f32),
            pltpu.VMEM((IB, R), jnp.int32),
            pltpu.VMEM((R, D), f32),
            pltpu.VMEM((R, D), f32),
            pltpu.SemaphoreType.DMA,
            pltpu.SemaphoreType.DMA,
            pltpu.SemaphoreType.DMA,
            pltpu.SemaphoreType.DMA,
        ],
    )
    s0, s1, c0, c1 = ka(input, idx2, zrow, zcnt, ones)

    kb = pl.kernel(
        _body_b,
        compiler_params=cparams,
        out_type=jax.ShapeDtypeStruct((E, D), f32),
        mesh=mesh,
        scratch_types=[
            pltpu.VMEM_SHARED((NPAD, D), f32),
            pltpu.VMEM((R, CW), f32),
            pltpu.VMEM((R, CW), f32),
            pltpu.VMEM((NFULL + NEX, R), jnp.int32),
            pltpu.VMEM((R, D), f32),
            pltpu.VMEM((R, D), f32),
            pltpu.SemaphoreType.DMA,
            pltpu.SemaphoreType.DMA,
            pltpu.SemaphoreType.DMA,
            pltpu.SemaphoreType.DMA,
            pltpu.SemaphoreType.DMA,
        ],
    )
    return kb(idx2, s0, s1, c0, c1)


# speedup vs baseline: 1.0702x; 1.0077x over previous
import jax
import jax.numpy as jnp
from jax import lax
from jax.experimental import pallas as pl
from jax.experimental.pallas import tpu as pltpu
from jax.experimental.pallas import tpu_sc as plsc

EPS = 1e-09
E = 320000
D = 128
N = 10000
NC = 2
NS = 16
NW = NC * NS
NPAD = 10240
SLAB = NPAD // NS
CW = 16
EPT = E // NW
R = 128
NFULL = EPT // R
TR = EPT - NFULL * R
NPAIR = NFULL // 2
NSLAB = SLAB // R


def _body_a(x_hbm, idx_hbm, zrow_hbm, zcnt_hbm, one_hbm,
            s0_hbm, s1_hbm, c0_hbm, c1_hbm,
            acc_sh, cnt_sh, zc_v, ones_v,
            idx0_v, idx1_v, idxt_v, rows0_v, rows1_v,
            ld0_s, ld1_s, sc0_s, sc1_s):
    c = lax.axis_index("c")
    s = lax.axis_index("s")
    idxs = (idx0_v, idx1_v)
    rows = (rows0_v, rows1_v)
    lds = (ld0_s, ld1_s)
    scs = (sc0_s, sc1_s)
    row0 = s * SLAB
    pltpu.sync_copy(zrow_hbm, rows0_v)
    pltpu.sync_copy(zcnt_hbm, zc_v)
    pltpu.sync_copy(one_hbm, ones_v)
    zds = []
    for j in range(NSLAB):
        zds.append(pltpu.async_copy(
            rows0_v, acc_sh.at[pl.ds(row0 + j * R, R), :], sc0_s))
        zds.append(pltpu.async_copy(
            zc_v, cnt_sh.at[pl.ds(row0 + j * R, R), :], sc1_s))
    for d in zds:
        d.wait()
    plsc.subcore_barrier()

    base = (c * NS + s) * EPT

    def start_load(off, b):
        pltpu.async_copy(idx_hbm.at[pl.ds(off, R)], idxs[b], lds[b])
        pltpu.async_copy(x_hbm.at[pl.ds(off, R), :], rows[b], lds[b])

    def wait_load(b):
        pltpu.make_async_copy(idx_hbm.at[pl.ds(0, R)], idxs[b], lds[b]).wait()
        pltpu.make_async_copy(x_hbm.at[pl.ds(0, R), :], rows[b], lds[b]).wait()

    start_load(base, 0)
    start_load(base + R, 1)

    def pair(i, carry):
        for b in range(2):
            ch = 2 * i + b
            wait_load(b)
            d1 = pltpu.async_copy(rows[b], acc_sh.at[idxs[b]], scs[b], add=True)
            d2 = pltpu.async_copy(ones_v, cnt_sh.at[idxs[b]], scs[b], add=True)
            d1.wait()
            d2.wait()

            @pl.when(ch + 2 < NFULL)
            def _():
                start_load(base + (ch + 2) * R, b)

        return carry

    lax.fori_loop(0, NPAIR, pair, 0)
    pltpu.sync_copy(idx_hbm.at[pl.ds(base + NFULL * R, TR)], idxt_v)
    pltpu.sync_copy(x_hbm.at[pl.ds(base + NFULL * R, TR), :],
                    rows0_v.at[pl.ds(0, TR), :])
    pltpu.sync_copy(rows0_v.at[pl.ds(0, TR), :], acc_sh.at[idxt_v], add=True)
    pltpu.sync_copy(ones_v.at[pl.ds(0, TR), :], cnt_sh.at[idxt_v], add=True)
    plsc.subcore_barrier()

    cbufs = (zc_v, ones_v)

    def dump_read(j, q):
        r0 = row0 + j * R
        pltpu.async_copy(acc_sh.at[pl.ds(r0, R), :], rows[q], lds[q])
        pltpu.async_copy(cnt_sh.at[pl.ds(r0, R), :], cbufs[q], lds[q])

    def dump_read_wait(q):
        pltpu.make_async_copy(acc_sh.at[pl.ds(0, R), :], rows[q], lds[q]).wait()
        pltpu.make_async_copy(cnt_sh.at[pl.ds(0, R), :], cbufs[q], lds[q]).wait()

    def dump_write_wait(q):
        pltpu.make_async_copy(rows[q], s0_hbm.at[pl.ds(0, R), :], scs[q]).wait()
        pltpu.make_async_copy(cbufs[q], c0_hbm.at[pl.ds(0, R), :], scs[q]).wait()

    dump_read(0, 0)
    for j in range(NSLAB):
        q = j % 2
        r0 = row0 + j * R
        dump_read_wait(q)
        if j + 1 < NSLAB:
            if j >= 1:
                dump_write_wait(q ^ 1)
            dump_read(j + 1, q ^ 1)

        @pl.when(c == 0)
        def _():
            pltpu.async_copy(rows[q], s0_hbm.at[pl.ds(r0, R), :], scs[q])
            pltpu.async_copy(cbufs[q], c0_hbm.at[pl.ds(r0, R), :], scs[q])

        @pl.when(c == 1)
        def _():
            pltpu.async_copy(rows[q], s1_hbm.at[pl.ds(r0, R), :], scs[q])
            pltpu.async_copy(cbufs[q], c1_hbm.at[pl.ds(r0, R), :], scs[q])

    dump_write_wait(0)
    dump_write_wait(1)


def _body_b(idx_hbm, s0_hbm, s1_hbm, c0_hbm, c1_hbm, out_hbm,
            pooled_sh, ca_v, cb_v,
            idx0_v, idx1_v, idxt_v, rows0_v, rows1_v, rowst_v,
            ld0_s, ld1_s, g_s, st0_s, st1_s):
    c = lax.axis_index("c")
    s = lax.axis_index("s")
    idxs = (idx0_v, idx1_v)
    rows = (rows0_v, rows1_v)
    lds = (ld0_s, ld1_s)
    sts = (st0_s, st1_s)
    row0 = s * SLAB

    HC = R // 2
    NH = SLAB // HC

    def comb_load(t, q):
        r0 = row0 + t * HC
        h = pl.ds(q * HC, HC)
        pltpu.async_copy(s0_hbm.at[pl.ds(r0, HC), :], rows0_v.at[h, :], lds[q])
        pltpu.async_copy(s1_hbm.at[pl.ds(r0, HC), :], rows1_v.at[h, :], lds[q])
        pltpu.async_copy(c0_hbm.at[pl.ds(r0, HC), :], ca_v.at[h, :], lds[q])
        pltpu.async_copy(c1_hbm.at[pl.ds(r0, HC), :], cb_v.at[h, :], lds[q])

    def comb_wait(q):
        h = pl.ds(q * HC, HC)
        pltpu.make_async_copy(s0_hbm.at[pl.ds(0, HC), :], rows0_v.at[h, :],
                              lds[q]).wait()
        pltpu.make_async_copy(s1_hbm.at[pl.ds(0, HC), :], rows1_v.at[h, :],
                              lds[q]).wait()
        pltpu.make_async_copy(c0_hbm.at[pl.ds(0, HC), :], ca_v.at[h, :],
                              lds[q]).wait()
        pltpu.make_async_copy(c1_hbm.at[pl.ds(0, HC), :], cb_v.at[h, :],
                              lds[q]).wait()

    comb_load(0, 0)

    def comb_pair(i, carry):
        for q in range(2):
            t = 2 * i + q
            comb_wait(q)

            @pl.when(t + 1 < NH)
            def _():
                comb_load(t + 1, q ^ 1)

            def nrow(r, cc):
                rr = q * HC + r
                sv = (ca_v[rr, pl.ds(0, 16)] + cb_v[rr, pl.ds(0, 16)]
                      + jnp.float32(EPS))
                scale = jnp.float32(1.0) / sv
                for k in range(8):
                    sl = pl.ds(k * 16, 16)
                    rows0_v[rr, sl] = (rows0_v[rr, sl] + rows1_v[rr, sl]) * scale
                return cc

            lax.fori_loop(0, HC, nrow, 0)
            pltpu.sync_copy(rows0_v.at[pl.ds(q * HC, HC), :],
                            pooled_sh.at[pl.ds(row0 + t * HC, HC), :])
        return carry

    lax.fori_loop(0, NH // 2, comb_pair, 0)
    plsc.subcore_barrier()

    base = (c * NS + s) * EPT

    def wait_idx(b):
        pltpu.make_async_copy(idx_hbm.at[pl.ds(0, R)], idxs[b], lds[b]).wait()

    def wait_store(b):
        pltpu.make_async_copy(rows[b], out_hbm.at[pl.ds(0, R), :], sts[b]).wait()

    pltpu.async_copy(idx_hbm.at[pl.ds(base, R)], idx0_v, ld0_s)
    pltpu.async_copy(idx_hbm.at[pl.ds(base + R, R)], idx1_v, ld1_s)

    def gpair(i, carry):
        for b in range(2):
            ch = 2 * i + b
            wait_idx(b)

            @pl.when(ch >= 2)
            def _():
                wait_store(b)

            g = pltpu.async_copy(pooled_sh.at[idxs[b]], rows[b], g_s)
            g.wait()
            pltpu.async_copy(rows[b], out_hbm.at[pl.ds(base + ch * R, R), :],
                             sts[b])

            @pl.when(ch + 2 < NFULL)
            def _():
                pltpu.async_copy(idx_hbm.at[pl.ds(base + (ch + 2) * R, R)],
                                 idxs[b], lds[b])

        return carry

    lax.fori_loop(0, NPAIR, gpair, 0)
    pltpu.sync_copy(idx_hbm.at[pl.ds(base + NFULL * R, TR)], idxt_v)
    pltpu.sync_copy(pooled_sh.at[idxt_v], rowst_v)
    pltpu.sync_copy(rowst_v, out_hbm.at[pl.ds(base + NFULL * R, TR), :])
    wait_store(0)
    wait_store(1)


def kernel(input, index):
    mesh = plsc.VectorSubcoreMesh(core_axis_name="c", subcore_axis_name="s",
                                  num_cores=NC, num_subcores=NS)
    f32 = jnp.float32
    zrow = jnp.zeros((R, D), f32)
    zcnt = jnp.zeros((R, CW), f32)
    ones = jnp.ones((R, CW), f32)

    cparams = pltpu.CompilerParams(use_tc_tiling_on_sc=False)
    ka = pl.kernel(
        _body_a,
        compiler_params=cparams,
        out_type=[jax.ShapeDtypeStruct((NPAD, D), f32),
                  jax.ShapeDtypeStruct((NPAD, D), f32),
                  jax.ShapeDtypeStruct((NPAD, CW), f32),
                  jax.ShapeDtypeStruct((NPAD, CW), f32)],
        mesh=mesh,
        scratch_types=[
            pltpu.VMEM_SHARED((NPAD, D), f32),
            pltpu.VMEM_SHARED((NPAD, CW), f32),
            pltpu.VMEM((R, CW), f32),
            pltpu.VMEM((R, CW), f32),
            pltpu.VMEM((R,), jnp.int32),
            pltpu.VMEM((R,), jnp.int32),
            pltpu.VMEM((TR,), jnp.int32),
            pltpu.VMEM((R, D), f32),
            pltpu.VMEM((R, D), f32),
            pltpu.SemaphoreType.DMA,
            pltpu.SemaphoreType.DMA,
            pltpu.SemaphoreType.DMA,
            pltpu.SemaphoreType.DMA,
        ],
    )
    s0, s1, c0, c1 = ka(input, index, zrow, zcnt, ones)

    kb = pl.kernel(
        _body_b,
        compiler_params=cparams,
        out_type=jax.ShapeDtypeStruct((E, D), f32),
        mesh=mesh,
        scratch_types=[
            pltpu.VMEM_SHARED((NPAD, D), f32),
            pltpu.VMEM((R, CW), f32),
            pltpu.VMEM((R, CW), f32),
            pltpu.VMEM((R,), jnp.int32),
            pltpu.VMEM((R,), jnp.int32),
            pltpu.VMEM((TR,), jnp.int32),
            pltpu.VMEM((R, D), f32),
            pltpu.VMEM((R, D), f32),
            pltpu.VMEM((TR, D), f32),
            pltpu.SemaphoreType.DMA,
            pltpu.SemaphoreType.DMA,
            pltpu.SemaphoreType.DMA,
            pltpu.SemaphoreType.DMA,
            pltpu.SemaphoreType.DMA,
        ],
    )
    return kb(index, s0, s1, c0, c1)
